# baseline (speedup 1.0000x reference)
import functools
import numpy as np
import jax
import jax.numpy as jnp
from jax import lax
from jax.experimental import pallas as pl
from jax.experimental.pallas import tpu as pltpu
from jax.experimental.pallas import tpu_sc as plsc

N = 10000
NPG = 200
B = 50
EPG = 3200
K = 500
KPG = 10
KP = 16
BK = B * KP
H = 64
DIN = 128
NC = 10


_NW = 32


def _adj_sc_body(src_hbm, dst_hbm, zeros_hbm, out_hbm, src_v, dst_v, acc_v):
    wid = lax.axis_index("s") * 2 + lax.axis_index("c")
    ones = jnp.full((16,), 1.0, jnp.float32)

    def do_graph(g):
        pltpu.sync_copy(zeros_hbm, acc_v)
        base_e = g * EPG
        pltpu.sync_copy(src_hbm.at[pl.ds(base_e, EPG)], src_v)
        pltpu.sync_copy(dst_hbm.at[pl.ds(base_e, EPG)], dst_v)
        base = g * (NPG * NPG + NPG)

        def body(i, carry):
            sl = pl.ds(i * 16, 16)
            idx = dst_v[sl] * NPG + src_v[sl] - base
            plsc.addupdate_scatter(acc_v, [idx], ones)
            return carry

        lax.fori_loop(0, EPG // 16, body, 0)
        pltpu.sync_copy(acc_v, out_hbm.at[pl.ds(g * NPG * NPG, NPG * NPG)])

    do_graph(wid)

    @pl.when(wid < B - _NW)
    def _():
        do_graph(wid + _NW)


_adj_sc = functools.partial(
    pl.kernel,
    mesh=plsc.VectorSubcoreMesh(core_axis_name="c", subcore_axis_name="s"),
    out_type=jax.ShapeDtypeStruct((N * NPG,), jnp.float32),
    scratch_types=[
        pltpu.VMEM((EPG,), jnp.int32),
        pltpu.VMEM((EPG,), jnp.int32),
        pltpu.VMEM((NPG * NPG,), jnp.float32),
    ],
    compiler_params=pltpu.CompilerParams(needs_layout_passes=False),
)(_adj_sc_body)


def _adj_body(src_ref, dst_ref, a_ref):
    g = pl.program_id(0)
    base = g * NPG
    src = src_ref[0] - base
    dst = dst_ref[0] - base
    rows = lax.broadcasted_iota(jnp.int32, (NPG, 1), 0)
    doh = (dst == rows).astype(jnp.bfloat16)
    soh = (src == rows).astype(jnp.bfloat16)
    a_ref[...] = lax.dot_general(
        doh, soh, (((1,), (1,)), ((), ())),
        preferred_element_type=jnp.float32)


def _net_body(h_ref, a_ref,
              wemb_ref, bemb_ref,
              ws1_ref, bs1_ref, g1_ref, be1_ref,
              ws2_ref, bs2_ref, g2_ref, be2_ref,
              ws3_ref, bs3_ref,
              wf_ref, bf_ref,
              wpp_ref, bpp_ref, wpre_ref, bpre_ref,
              wd1_ref, bd1_ref, wd2_ref, bd2_ref, wd3_ref, bd3_ref,
              wpred_ref, bpred_ref,
              o_ref,
              sx0, sx1, sx2, sc, sdiv, shp, sadj):
    f32 = jnp.float32
    ones_hh = jnp.ones((H, H), f32)
    ones_1n = jnp.ones((1, N), f32)

    deg_rep = jnp.dot(a_ref[...], jnp.ones((NPG, H), f32),
                      preferred_element_type=f32)
    sdiv[...] = 1.0 / jnp.maximum(deg_rep, 1.0)

    sx0[...] = jnp.dot(h_ref[...], wemb_ref[...],
                       preferred_element_type=f32) + bemb_ref[...]

    def agg(x_scr):
        def body(g, carry):
            sl = pl.ds(g * NPG, NPG)
            sc[sl, :] = jnp.dot(a_ref[sl, :], x_scr[sl, :],
                                preferred_element_type=f32)
            return carry
        lax.fori_loop(0, B, body, 0, unroll=5)
        sc[...] = sc[...] * sdiv[...]

    def rrsqrt(x2):
        return jnp.minimum(lax.rsqrt(x2), 1e12)

    def sage(x_scr, out_scr, w_ref, b_ref, act, bn, residual):
        agg(x_scr)
        x = x_scr[...]
        c = sc[...]
        w = w_ref[...]
        hh = (jnp.dot(x, w[:H, :], preferred_element_type=f32)
              + jnp.dot(c, w[H:, :], preferred_element_type=f32)
              + b_ref[...])
        nrm2 = jnp.dot(hh * hh, ones_hh, preferred_element_type=f32)
        hh = hh * rrsqrt(nrm2)
        if act:
            hh = jnp.maximum(hh, 0.0)
        if bn is not None:
            gr, ber = bn
            mu = jnp.dot(ones_1n, hh, preferred_element_type=f32) * (1.0 / N)
            ex2 = jnp.dot(ones_1n, hh * hh,
                          preferred_element_type=f32) * (1.0 / N)
            var = ex2 - mu * mu
            scale = gr[...] / jnp.sqrt(var + 1e-5)
            hh = hh * scale + (ber[...] - mu * scale)
        if residual:
            hh = x + hh
        out_scr[...] = hh

    sage(sx0, sx1, ws1_ref, bs1_ref, True, (g1_ref, be1_ref), True)
    sage(sx1, sx2, ws2_ref, bs2_ref, True, (g2_ref, be2_ref), True)
    sage(sx2, sx0, ws3_ref, bs3_ref, False, None, True)

    agg(sx0)

    wf = wf_ref[...]
    ff = (jnp.dot(sx0[...], wf[:H, :], preferred_element_type=f32)
          + jnp.dot(sc[...], wf[H:, :], preferred_element_type=f32)
          + bf_ref[...])
    fn2 = jnp.dot(ff * ff, ones_hh, preferred_element_type=f32)
    sx1[...] = jnp.maximum(ff * rrsqrt(fn2), 0.0)

    wpp = wpp_ref[...]
    gram = lax.dot_general(wpp, wpp, (((1,), (1,)), ((), ())),
                           preferred_element_type=f32)
    ones_hk = jnp.ones((H, KP), f32)
    wb = lax.dot_general(wpp, bpp_ref[...], (((1,), (1,)), ((), ())),
                         preferred_element_type=f32)
    wbk = wb * jnp.ones((1, KP), f32)
    bb = jnp.sum(bpp_ref[...] ** 2)
    gemb = sx0[...]
    cg = sc[...]
    t1 = (jnp.dot(gemb, gram[:H, :H], preferred_element_type=f32)
          + jnp.dot(cg, gram[H:, :H], preferred_element_type=f32))
    t2 = (jnp.dot(gemb, gram[:H, H:], preferred_element_type=f32)
          + jnp.dot(cg, gram[H:, H:], preferred_element_type=f32))
    lin = (jnp.dot(gemb, wbk[:H, :], preferred_element_type=f32)
           + jnp.dot(cg, wbk[H:, :], preferred_element_type=f32))
    nsq = (jnp.dot(t1 * gemb, ones_hk, preferred_element_type=f32)
           + jnp.dot(t2 * cg, ones_hk, preferred_element_type=f32)
           + 2.0 * lin + bb)
    sx2[:, 0:KP] = jnp.maximum(nsq, 0.0)

    lane = lax.broadcasted_iota(jnp.int32, (NPG, KP), 1)
    valid = (lane < KPG).astype(f32)

    def pool_body(g, carry):
        sl = pl.ds(g * NPG, NPG)
        xg = sx0[sl, :]
        cgg = sc[sl, :]
        wt = wpre_ref[g]
        bt = bpre_ref[g]
        hh = (jnp.dot(xg, wt[:H, :], preferred_element_type=f32)
              + jnp.dot(cgg, wt[H:, :], preferred_element_type=f32) + bt)
        rn = jnp.minimum(lax.rsqrt(sx2[sl, 0:KP]), 1e12)
        ex = jnp.exp(jnp.maximum(hh, 0.0) * rn) * valid
        zin = jnp.sum(ex, axis=1, keepdims=True)
        zfull = zin + float(K - KPG)
        s = ex / (zin + 1e-13 * zfull)
        featg = sx1[sl, :]
        hp = lax.dot_general(s, featg, (((0,), (0,)), ((), ())),
                             preferred_element_type=f32)
        shp[pl.ds(g * KP, KP), :] = hp
        asg = jnp.dot(a_ref[sl, :], s, preferred_element_type=f32)
        adj = lax.dot_general(s, asg, (((0,), (0,)), ((), ())),
                              preferred_element_type=f32)
        sadj[pl.ds(g * KP, KP), :] = adj
        return carry

    lax.fori_loop(0, B, pool_body, 0, unroll=2)

    kk = lax.broadcasted_iota(jnp.int32, (KP, BK), 0)
    cc = lax.broadcasted_iota(jnp.int32, (KP, BK), 1)
    pmat = (cc % KP == kk).astype(f32)
    rr = lax.broadcasted_iota(jnp.int32, (BK, BK), 0)
    cb = lax.broadcasted_iota(jnp.int32, (BK, BK), 1)
    blockmask = (rr // KP == cb // KP).astype(f32)
    bd = jnp.dot(sadj[...], pmat, preferred_element_type=f32) * blockmask

    x = shp[...] * float(np.sqrt(1.0 / KPG))
    for wd_ref, bdr in ((wd1_ref, bd1_ref), (wd2_ref, bd2_ref),
                        (wd3_ref, bd3_ref)):
        hk = jnp.dot(bd, x, preferred_element_type=f32)
        hk = jnp.dot(hk, wd_ref[...], preferred_element_type=f32) + bdr[...]
        nrm2 = jnp.dot(hk * hk, ones_hh, preferred_element_type=f32)
        hk = jnp.maximum(hk * rrsqrt(nrm2), 0.0)
        hk = x + hk
        sums = jnp.dot(pmat, hk, preferred_element_type=f32)
        mu = jnp.sum(sums, axis=1, keepdims=True) / (B * H)
        sq = jnp.dot(pmat, hk * hk, preferred_element_type=f32)
        ex2 = jnp.sum(sq, axis=1, keepdims=True) / (B * H)
        var = ex2 - mu * mu
        onesh = jnp.ones((1, H), f32)
        mu_b = lax.dot_general(pmat, mu * onesh, (((0,), (0,)), ((), ())),
                               preferred_element_type=f32)
        rs_b = lax.dot_general(pmat, lax.rsqrt(var + 1e-5) * onesh,
                               (((0,), (0,)), ((), ())),
                               preferred_element_type=f32)
        x = (hk - mu_b) * rs_b

    gg = lax.broadcasted_iota(jnp.int32, (B, BK), 0)
    rq = lax.broadcasted_iota(jnp.int32, (B, BK), 1)
    q = ((rq // KP == gg) & (rq % KP < KPG)).astype(f32)
    readout = jnp.dot(q, x, preferred_element_type=f32)
    o_ref[...] = (jnp.dot(readout, wpred_ref[...],
                          preferred_element_type=f32) + bpred_ref[...])


def kernel(h, edge_index, e, snorm_n, snorm_e, params):
    p = params
    src = edge_index[0].astype(jnp.int32)
    dst = edge_index[1].astype(jnp.int32)
    zeros_blk = jnp.zeros((NPG * NPG,), jnp.float32)
    adj = _adj_sc(src, dst, zeros_blk).reshape(N, NPG)

    wpre = p['W_dpp'].T.reshape(B, KPG, 2 * H).transpose(0, 2, 1)
    wpre = jnp.pad(wpre, ((0, 0), (0, 0), (0, KP - KPG)))
    bpre = jnp.pad(p['b_dpp'].reshape(B, 1, KPG),
                   ((0, 0), (0, 0), (0, KP - KPG)))

    f32 = jnp.float32
    out = pl.pallas_call(
        _net_body,
        out_shape=jax.ShapeDtypeStruct((B, NC), f32),
        scratch_shapes=[
            pltpu.VMEM((N, H), f32),
            pltpu.VMEM((N, H), f32),
            pltpu.VMEM((N, H), f32),
            pltpu.VMEM((N, H), f32),
            pltpu.VMEM((N, H), f32),
            pltpu.VMEM((BK, H), f32),
            pltpu.VMEM((BK, KP), f32),
        ],
    )(h, adj,
      p['W_emb'], p['b_emb'].reshape(1, H),
      p['W_s1'], p['b_s1'].reshape(1, H), p['g1'].reshape(1, H), p['be1'].reshape(1, H),
      p['W_s2'], p['b_s2'].reshape(1, H), p['g2'].reshape(1, H), p['be2'].reshape(1, H),
      p['W_s3'], p['b_s3'].reshape(1, H),
      p['W_dpf'], p['b_dpf'].reshape(1, H),
      p['W_dpp'], p['b_dpp'].reshape(1, K), wpre, bpre,
      p['W_d1'], p['b_d1'].reshape(1, H),
      p['W_d2'], p['b_d2'].reshape(1, H),
      p['W_d3'], p['b_d3'].reshape(1, H),
      p['W_pred'], p['b_pred'].reshape(1, NC))
    return out


# speedup vs baseline: 1.0016x; 1.0016x over previous
import functools
import numpy as np
import jax
import jax.numpy as jnp
from jax import lax
from jax.experimental import pallas as pl
from jax.experimental.pallas import tpu as pltpu
from jax.experimental.pallas import tpu_sc as plsc

N = 10000
NPG = 200
B = 50
EPG = 3200
K = 500
KPG = 10
KP = 16
BK = B * KP
H = 64
DIN = 128
NC = 10


_NW = 32


def _adj_sc_body(src_hbm, dst_hbm, zeros_hbm, out_hbm, src_v, dst_v, acc_v):
    wid = lax.axis_index("s") * 2 + lax.axis_index("c")
    ones = jnp.full((16,), 1.0, jnp.float32)

    def do_graph(g):
        pltpu.sync_copy(zeros_hbm, acc_v)
        base_e = g * EPG
        pltpu.sync_copy(src_hbm.at[pl.ds(base_e, EPG)], src_v)
        pltpu.sync_copy(dst_hbm.at[pl.ds(base_e, EPG)], dst_v)
        base = g * (NPG * NPG + NPG)

        def body(i, carry):
            sl = pl.ds(i * 16, 16)
            idx = dst_v[sl] * NPG + src_v[sl] - base
            plsc.addupdate_scatter(acc_v, [idx], ones)
            return carry

        lax.fori_loop(0, EPG // 16, body, 0, unroll=4)
        pltpu.sync_copy(acc_v, out_hbm.at[pl.ds(g * NPG * NPG, NPG * NPG)])

    do_graph(wid)

    @pl.when(wid < B - _NW)
    def _():
        do_graph(wid + _NW)


_adj_sc = functools.partial(
    pl.kernel,
    mesh=plsc.VectorSubcoreMesh(core_axis_name="c", subcore_axis_name="s"),
    out_type=jax.ShapeDtypeStruct((N * NPG,), jnp.float32),
    scratch_types=[
        pltpu.VMEM((EPG,), jnp.int32),
        pltpu.VMEM((EPG,), jnp.int32),
        pltpu.VMEM((NPG * NPG,), jnp.float32),
    ],
    compiler_params=pltpu.CompilerParams(needs_layout_passes=False),
)(_adj_sc_body)


def _adj_body(src_ref, dst_ref, a_ref):
    g = pl.program_id(0)
    base = g * NPG
    src = src_ref[0] - base
    dst = dst_ref[0] - base
    rows = lax.broadcasted_iota(jnp.int32, (NPG, 1), 0)
    doh = (dst == rows).astype(jnp.bfloat16)
    soh = (src == rows).astype(jnp.bfloat16)
    a_ref[...] = lax.dot_general(
        doh, soh, (((1,), (1,)), ((), ())),
        preferred_element_type=jnp.float32)


def _net_body(h_ref, a_ref,
              wemb_ref, bemb_ref,
              ws1_ref, bs1_ref, g1_ref, be1_ref,
              ws2_ref, bs2_ref, g2_ref, be2_ref,
              ws3_ref, bs3_ref,
              wf_ref, bf_ref,
              wpp_ref, bpp_ref, wpre_ref, bpre_ref,
              wd1_ref, bd1_ref, wd2_ref, bd2_ref, wd3_ref, bd3_ref,
              wpred_ref, bpred_ref,
              o_ref,
              sx0, sx1, sx2, sc, sdiv, shp, sadj):
    f32 = jnp.float32
    ones_hh = jnp.ones((H, H), f32)
    ones_1n = jnp.ones((1, N), f32)

    deg_rep = jnp.dot(a_ref[...], jnp.ones((NPG, H), f32),
                      preferred_element_type=f32)
    sdiv[...] = 1.0 / jnp.maximum(deg_rep, 1.0)

    sx0[...] = jnp.dot(h_ref[...], wemb_ref[...],
                       preferred_element_type=f32) + bemb_ref[...]

    def agg(x_scr):
        def body(g, carry):
            sl = pl.ds(g * NPG, NPG)
            sc[sl, :] = jnp.dot(a_ref[sl, :], x_scr[sl, :],
                                preferred_element_type=f32)
            return carry
        lax.fori_loop(0, B, body, 0, unroll=5)
        sc[...] = sc[...] * sdiv[...]

    def rrsqrt(x2):
        return jnp.minimum(lax.rsqrt(x2), 1e12)

    def sage(x_scr, out_scr, w_ref, b_ref, act, bn, residual):
        agg(x_scr)
        x = x_scr[...]
        c = sc[...]
        w = w_ref[...]
        hh = (jnp.dot(x, w[:H, :], preferred_element_type=f32)
              + jnp.dot(c, w[H:, :], preferred_element_type=f32)
              + b_ref[...])
        nrm2 = jnp.dot(hh * hh, ones_hh, preferred_element_type=f32)
        hh = hh * rrsqrt(nrm2)
        if act:
            hh = jnp.maximum(hh, 0.0)
        if bn is not None:
            gr, ber = bn
            mu = jnp.dot(ones_1n, hh, preferred_element_type=f32) * (1.0 / N)
            ex2 = jnp.dot(ones_1n, hh * hh,
                          preferred_element_type=f32) * (1.0 / N)
            var = ex2 - mu * mu
            scale = gr[...] / jnp.sqrt(var + 1e-5)
            hh = hh * scale + (ber[...] - mu * scale)
        if residual:
            hh = x + hh
        out_scr[...] = hh

    sage(sx0, sx1, ws1_ref, bs1_ref, True, (g1_ref, be1_ref), True)
    sage(sx1, sx2, ws2_ref, bs2_ref, True, (g2_ref, be2_ref), True)
    sage(sx2, sx0, ws3_ref, bs3_ref, False, None, True)

    agg(sx0)

    wf = wf_ref[...]
    ff = (jnp.dot(sx0[...], wf[:H, :], preferred_element_type=f32)
          + jnp.dot(sc[...], wf[H:, :], preferred_element_type=f32)
          + bf_ref[...])
    fn2 = jnp.dot(ff * ff, ones_hh, preferred_element_type=f32)
    sx1[...] = jnp.maximum(ff * rrsqrt(fn2), 0.0)

    wpp = wpp_ref[...]
    gram = lax.dot_general(wpp, wpp, (((1,), (1,)), ((), ())),
                           preferred_element_type=f32)
    ones_hk = jnp.ones((H, KP), f32)
    wb = lax.dot_general(wpp, bpp_ref[...], (((1,), (1,)), ((), ())),
                         preferred_element_type=f32)
    wbk = wb * jnp.ones((1, KP), f32)
    bb = jnp.sum(bpp_ref[...] ** 2)
    gemb = sx0[...]
    cg = sc[...]
    t1 = (jnp.dot(gemb, gram[:H, :H], preferred_element_type=f32)
          + jnp.dot(cg, gram[H:, :H], preferred_element_type=f32))
    t2 = (jnp.dot(gemb, gram[:H, H:], preferred_element_type=f32)
          + jnp.dot(cg, gram[H:, H:], preferred_element_type=f32))
    lin = (jnp.dot(gemb, wbk[:H, :], preferred_element_type=f32)
           + jnp.dot(cg, wbk[H:, :], preferred_element_type=f32))
    nsq = (jnp.dot(t1 * gemb, ones_hk, preferred_element_type=f32)
           + jnp.dot(t2 * cg, ones_hk, preferred_element_type=f32)
           + 2.0 * lin + bb)
    sx2[:, 0:KP] = jnp.maximum(nsq, 0.0)

    lane = lax.broadcasted_iota(jnp.int32, (NPG, KP), 1)
    valid = (lane < KPG).astype(f32)

    def pool_body(g, carry):
        sl = pl.ds(g * NPG, NPG)
        xg = sx0[sl, :]
        cgg = sc[sl, :]
        wt = wpre_ref[g]
        bt = bpre_ref[g]
        hh = (jnp.dot(xg, wt[:H, :], preferred_element_type=f32)
              + jnp.dot(cgg, wt[H:, :], preferred_element_type=f32) + bt)
        rn = jnp.minimum(lax.rsqrt(sx2[sl, 0:KP]), 1e12)
        ex = jnp.exp(jnp.maximum(hh, 0.0) * rn) * valid
        zin = jnp.sum(ex, axis=1, keepdims=True)
        zfull = zin + float(K - KPG)
        s = ex / (zin + 1e-13 * zfull)
        st = jnp.swapaxes(s, 0, 1)
        featg = sx1[sl, :]
        shp[pl.ds(g * KP, KP), :] = jnp.dot(st, featg,
                                            preferred_element_type=f32)
        asg = jnp.dot(a_ref[sl, :], s, preferred_element_type=f32)
        sadj[pl.ds(g * KP, KP), :] = jnp.dot(st, asg,
                                             preferred_element_type=f32)
        return carry

    lax.fori_loop(0, B, pool_body, 0, unroll=2)

    kk = lax.broadcasted_iota(jnp.int32, (KP, BK), 0)
    cc = lax.broadcasted_iota(jnp.int32, (KP, BK), 1)
    pmat = (cc % KP == kk).astype(f32)
    rr = lax.broadcasted_iota(jnp.int32, (BK, BK), 0)
    cb = lax.broadcasted_iota(jnp.int32, (BK, BK), 1)
    blockmask = (rr // KP == cb // KP).astype(f32)
    bd = jnp.dot(sadj[...], pmat, preferred_element_type=f32) * blockmask

    x = shp[...] * float(np.sqrt(1.0 / KPG))
    for wd_ref, bdr in ((wd1_ref, bd1_ref), (wd2_ref, bd2_ref),
                        (wd3_ref, bd3_ref)):
        hk = jnp.dot(bd, x, preferred_element_type=f32)
        hk = jnp.dot(hk, wd_ref[...], preferred_element_type=f32) + bdr[...]
        nrm2 = jnp.dot(hk * hk, ones_hh, preferred_element_type=f32)
        hk = jnp.maximum(hk * rrsqrt(nrm2), 0.0)
        hk = x + hk
        sums = jnp.dot(pmat, hk, preferred_element_type=f32)
        mu = jnp.sum(sums, axis=1, keepdims=True) / (B * H)
        sq = jnp.dot(pmat, hk * hk, preferred_element_type=f32)
        ex2 = jnp.sum(sq, axis=1, keepdims=True) / (B * H)
        var = ex2 - mu * mu
        onesh = jnp.ones((1, H), f32)
        mu_b = lax.dot_general(pmat, mu * onesh, (((0,), (0,)), ((), ())),
                               preferred_element_type=f32)
        rs_b = lax.dot_general(pmat, lax.rsqrt(var + 1e-5) * onesh,
                               (((0,), (0,)), ((), ())),
                               preferred_element_type=f32)
        x = (hk - mu_b) * rs_b

    gg = lax.broadcasted_iota(jnp.int32, (B, BK), 0)
    rq = lax.broadcasted_iota(jnp.int32, (B, BK), 1)
    q = ((rq // KP == gg) & (rq % KP < KPG)).astype(f32)
    readout = jnp.dot(q, x, preferred_element_type=f32)
    o_ref[...] = (jnp.dot(readout, wpred_ref[...],
                          preferred_element_type=f32) + bpred_ref[...])


def kernel(h, edge_index, e, snorm_n, snorm_e, params):
    p = params
    src = edge_index[0].astype(jnp.int32)
    dst = edge_index[1].astype(jnp.int32)
    zeros_blk = jnp.zeros((NPG * NPG,), jnp.float32)
    adj = _adj_sc(src, dst, zeros_blk).reshape(N, NPG)

    wpre = p['W_dpp'].T.reshape(B, KPG, 2 * H).transpose(0, 2, 1)
    wpre = jnp.pad(wpre, ((0, 0), (0, 0), (0, KP - KPG)))
    bpre = jnp.pad(p['b_dpp'].reshape(B, 1, KPG),
                   ((0, 0), (0, 0), (0, KP - KPG)))

    f32 = jnp.float32
    out = pl.pallas_call(
        _net_body,
        out_shape=jax.ShapeDtypeStruct((B, NC), f32),
        scratch_shapes=[
            pltpu.VMEM((N, H), f32),
            pltpu.VMEM((N, H), f32),
            pltpu.VMEM((N, H), f32),
            pltpu.VMEM((N, H), f32),
            pltpu.VMEM((N, H), f32),
            pltpu.VMEM((BK, H), f32),
            pltpu.VMEM((BK, KP), f32),
        ],
    )(h, adj,
      p['W_emb'], p['b_emb'].reshape(1, H),
      p['W_s1'], p['b_s1'].reshape(1, H), p['g1'].reshape(1, H), p['be1'].reshape(1, H),
      p['W_s2'], p['b_s2'].reshape(1, H), p['g2'].reshape(1, H), p['be2'].reshape(1, H),
      p['W_s3'], p['b_s3'].reshape(1, H),
      p['W_dpf'], p['b_dpf'].reshape(1, H),
      p['W_dpp'], p['b_dpp'].reshape(1, K), wpre, bpre,
      p['W_d1'], p['b_d1'].reshape(1, H),
      p['W_d2'], p['b_d2'].reshape(1, H),
      p['W_d3'], p['b_d3'].reshape(1, H),
      p['W_pred'], p['b_pred'].reshape(1, NC))
    return out


# speedup vs baseline: 1.0259x; 1.0243x over previous
import functools
import numpy as np
import jax
import jax.numpy as jnp
from jax import lax
from jax.experimental import pallas as pl
from jax.experimental.pallas import tpu as pltpu
from jax.experimental.pallas import tpu_sc as plsc

N = 10000
NPG = 200
B = 50
EPG = 3200
K = 500
KPG = 10
KP = 16
BK = B * KP
H = 64
DIN = 128
NC = 10


_NW = 32


def _adj_sc_body(src_hbm, dst_hbm, zeros_hbm, out_hbm, src_v, dst_v, acc_v):
    wid = lax.axis_index("s") * 2 + lax.axis_index("c")
    ones = jnp.full((16,), 1.0, jnp.float32)

    def do_graph(g):
        pltpu.sync_copy(zeros_hbm, acc_v)
        base_e = g * EPG
        pltpu.sync_copy(src_hbm.at[pl.ds(base_e, EPG)], src_v)
        pltpu.sync_copy(dst_hbm.at[pl.ds(base_e, EPG)], dst_v)
        base = g * (NPG * NPG + NPG)

        def body(i, carry):
            sl = pl.ds(i * 16, 16)
            idx = dst_v[sl] * NPG + src_v[sl] - base
            plsc.addupdate_scatter(acc_v, [idx], ones)
            return carry

        lax.fori_loop(0, EPG // 16, body, 0, unroll=4)
        pltpu.sync_copy(acc_v, out_hbm.at[pl.ds(g * NPG * NPG, NPG * NPG)])

    do_graph(wid)

    @pl.when(wid < B - _NW)
    def _():
        do_graph(wid + _NW)


def _adj_sc(src, dst, zeros_blk):
    k = pl.kernel(
        _adj_sc_body,
        mesh=plsc.VectorSubcoreMesh(core_axis_name="c", subcore_axis_name="s"),
        out_type=jax.ShapeDtypeStruct((N * NPG,), jnp.float32),
        scratch_types=[
            pltpu.VMEM((EPG,), jnp.int32),
            pltpu.VMEM((EPG,), jnp.int32),
            pltpu.VMEM((NPG * NPG,), jnp.float32),
        ],
        compiler_params=pltpu.CompilerParams(needs_layout_passes=False),
    )
    return k(src, dst, zeros_blk)


def _adj_body(src_ref, dst_ref, a_ref):
    g = pl.program_id(0)
    base = g * NPG
    src = src_ref[0] - base
    dst = dst_ref[0] - base
    rows = lax.broadcasted_iota(jnp.int32, (NPG, 1), 0)
    doh = (dst == rows).astype(jnp.bfloat16)
    soh = (src == rows).astype(jnp.bfloat16)
    a_ref[...] = lax.dot_general(
        doh, soh, (((1,), (1,)), ((), ())),
        preferred_element_type=jnp.float32)


def _net_body(h_ref, a_ref,
              wemb_ref, bemb_ref,
              ws1_ref, bs1_ref, g1_ref, be1_ref,
              ws2_ref, bs2_ref, g2_ref, be2_ref,
              ws3_ref, bs3_ref,
              wf_ref, bf_ref,
              wpp_ref, bpp_ref, wpre_ref, bpre_ref,
              wd1_ref, bd1_ref, wd2_ref, bd2_ref, wd3_ref, bd3_ref,
              wpred_ref, bpred_ref,
              o_ref,
              sx0, sx1, sx2, sdiv, shp, sadj):
    f32 = jnp.float32
    ones_hh = jnp.ones((H, H), f32)
    ones_1n = jnp.ones((1, N), f32)

    deg_rep = jnp.dot(a_ref[...], jnp.ones((NPG, H), f32),
                      preferred_element_type=f32)
    sdiv[...] = 1.0 / jnp.maximum(deg_rep, 1.0)

    sx0[:, 0:H] = jnp.dot(h_ref[...], wemb_ref[...],
                          preferred_element_type=f32) + bemb_ref[...]

    def agg(x_scr):
        def body(g, carry):
            sl = pl.ds(g * NPG, NPG)
            x_scr[sl, H:2 * H] = jnp.dot(a_ref[sl, :], x_scr[sl, 0:H],
                                         preferred_element_type=f32)
            return carry
        lax.fori_loop(0, B, body, 0, unroll=5)
        x_scr[:, H:2 * H] = x_scr[:, H:2 * H] * sdiv[...]

    def rrsqrt(x2):
        return jnp.minimum(lax.rsqrt(x2), 1e12)

    def sage(x_scr, out_scr, w_ref, b_ref, act, bn, residual):
        agg(x_scr)
        hh = jnp.dot(x_scr[...], w_ref[...],
                     preferred_element_type=f32) + b_ref[...]
        nrm2 = jnp.dot(hh * hh, ones_hh, preferred_element_type=f32)
        hh = hh * rrsqrt(nrm2)
        if act:
            hh = jnp.maximum(hh, 0.0)
        if bn is not None:
            gr, ber = bn
            mu = jnp.dot(ones_1n, hh, preferred_element_type=f32) * (1.0 / N)
            ex2 = jnp.dot(ones_1n, hh * hh,
                          preferred_element_type=f32) * (1.0 / N)
            var = ex2 - mu * mu
            scale = gr[...] / jnp.sqrt(var + 1e-5)
            hh = hh * scale + (ber[...] - mu * scale)
        if residual:
            hh = x_scr[:, 0:H] + hh
        out_scr[:, 0:H] = hh

    sage(sx0, sx1, ws1_ref, bs1_ref, True, (g1_ref, be1_ref), True)
    sage(sx1, sx2, ws2_ref, bs2_ref, True, (g2_ref, be2_ref), True)
    sage(sx2, sx0, ws3_ref, bs3_ref, False, None, True)

    agg(sx0)

    ff = jnp.dot(sx0[...], wf_ref[...],
                 preferred_element_type=f32) + bf_ref[...]
    fn2 = jnp.dot(ff * ff, ones_hh, preferred_element_type=f32)
    sx1[:, 0:H] = jnp.maximum(ff * rrsqrt(fn2), 0.0)

    wpp = wpp_ref[...]
    gram = lax.dot_general(wpp, wpp, (((1,), (1,)), ((), ())),
                           preferred_element_type=f32)
    ones_hk = jnp.ones((2 * H, KP), f32)
    wb = lax.dot_general(wpp, bpp_ref[...], (((1,), (1,)), ((), ())),
                         preferred_element_type=f32)
    wbk = wb * jnp.ones((1, KP), f32)
    bb = jnp.sum(bpp_ref[...] ** 2)
    zb = sx0[...]
    t = jnp.dot(zb, gram, preferred_element_type=f32)
    lin = jnp.dot(zb, wbk, preferred_element_type=f32)
    nsq = (jnp.dot(t * zb, ones_hk, preferred_element_type=f32)
           + 2.0 * lin + bb)
    sx2[:, 0:KP] = jnp.maximum(nsq, 0.0)

    lane = lax.broadcasted_iota(jnp.int32, (NPG, KP), 1)
    valid = (lane < KPG).astype(f32)

    def pool_body(g, carry):
        sl = pl.ds(g * NPG, NPG)
        zg = sx0[sl, :]
        wt = wpre_ref[g]
        bt = bpre_ref[g]
        hh = jnp.dot(zg, wt, preferred_element_type=f32) + bt
        rn = jnp.minimum(lax.rsqrt(sx2[sl, 0:KP]), 1e12)
        ex = jnp.exp(jnp.maximum(hh, 0.0) * rn) * valid
        zin = jnp.sum(ex, axis=1, keepdims=True)
        zfull = zin + float(K - KPG)
        s = ex / (zin + 1e-13 * zfull)
        st = jnp.swapaxes(s, 0, 1)
        featg = sx1[sl, 0:H]
        shp[pl.ds(g * KP, KP), :] = jnp.dot(st, featg,
                                            preferred_element_type=f32)
        asg = jnp.dot(a_ref[sl, :], s, preferred_element_type=f32)
        sadj[pl.ds(g * KP, KP), :] = jnp.dot(st, asg,
                                             preferred_element_type=f32)
        return carry

    lax.fori_loop(0, B, pool_body, 0, unroll=2)

    kk = lax.broadcasted_iota(jnp.int32, (KP, BK), 0)
    cc = lax.broadcasted_iota(jnp.int32, (KP, BK), 1)
    pmat = (cc % KP == kk).astype(f32)
    rr = lax.broadcasted_iota(jnp.int32, (BK, BK), 0)
    cb = lax.broadcasted_iota(jnp.int32, (BK, BK), 1)
    blockmask = (rr // KP == cb // KP).astype(f32)
    bd = jnp.dot(sadj[...], pmat, preferred_element_type=f32) * blockmask

    x = shp[...] * float(np.sqrt(1.0 / KPG))
    for wd_ref, bdr in ((wd1_ref, bd1_ref), (wd2_ref, bd2_ref),
                        (wd3_ref, bd3_ref)):
        hk = jnp.dot(bd, x, preferred_element_type=f32)
        hk = jnp.dot(hk, wd_ref[...], preferred_element_type=f32) + bdr[...]
        nrm2 = jnp.dot(hk * hk, ones_hh, preferred_element_type=f32)
        hk = jnp.maximum(hk * rrsqrt(nrm2), 0.0)
        hk = x + hk
        sums = jnp.dot(pmat, hk, preferred_element_type=f32)
        mu = jnp.sum(sums, axis=1, keepdims=True) / (B * H)
        sq = jnp.dot(pmat, hk * hk, preferred_element_type=f32)
        ex2 = jnp.sum(sq, axis=1, keepdims=True) / (B * H)
        var = ex2 - mu * mu
        onesh = jnp.ones((1, H), f32)
        mu_b = lax.dot_general(pmat, mu * onesh, (((0,), (0,)), ((), ())),
                               preferred_element_type=f32)
        rs_b = lax.dot_general(pmat, lax.rsqrt(var + 1e-5) * onesh,
                               (((0,), (0,)), ((), ())),
                               preferred_element_type=f32)
        x = (hk - mu_b) * rs_b

    gg = lax.broadcasted_iota(jnp.int32, (B, BK), 0)
    rq = lax.broadcasted_iota(jnp.int32, (B, BK), 1)
    q = ((rq // KP == gg) & (rq % KP < KPG)).astype(f32)
    readout = jnp.dot(q, x, preferred_element_type=f32)
    o_ref[...] = (jnp.dot(readout, wpred_ref[...],
                          preferred_element_type=f32) + bpred_ref[...])


def kernel(h, edge_index, e, snorm_n, snorm_e, params):
    p = params
    src = edge_index[0].astype(jnp.int32)
    dst = edge_index[1].astype(jnp.int32)
    zeros_blk = jnp.zeros((NPG * NPG,), jnp.float32)
    adj = _adj_sc(src, dst, zeros_blk).reshape(N, NPG)

    wpre = p['W_dpp'].T.reshape(B, KPG, 2 * H).transpose(0, 2, 1)
    wpre = jnp.pad(wpre, ((0, 0), (0, 0), (0, KP - KPG)))
    bpre = jnp.pad(p['b_dpp'].reshape(B, 1, KPG),
                   ((0, 0), (0, 0), (0, KP - KPG)))

    f32 = jnp.float32
    out = pl.pallas_call(
        _net_body,
        out_shape=jax.ShapeDtypeStruct((B, NC), f32),
        scratch_shapes=[
            pltpu.VMEM((N, 2 * H), f32),
            pltpu.VMEM((N, 2 * H), f32),
            pltpu.VMEM((N, 2 * H), f32),
            pltpu.VMEM((N, H), f32),
            pltpu.VMEM((BK, H), f32),
            pltpu.VMEM((BK, KP), f32),
        ],
    )(h, adj,
      p['W_emb'], p['b_emb'].reshape(1, H),
      p['W_s1'], p['b_s1'].reshape(1, H), p['g1'].reshape(1, H), p['be1'].reshape(1, H),
      p['W_s2'], p['b_s2'].reshape(1, H), p['g2'].reshape(1, H), p['be2'].reshape(1, H),
      p['W_s3'], p['b_s3'].reshape(1, H),
      p['W_dpf'], p['b_dpf'].reshape(1, H),
      p['W_dpp'], p['b_dpp'].reshape(1, K), wpre, bpre,
      p['W_d1'], p['b_d1'].reshape(1, H),
      p['W_d2'], p['b_d2'].reshape(1, H),
      p['W_d3'], p['b_d3'].reshape(1, H),
      p['W_pred'], p['b_pred'].reshape(1, NC))
    return out
